# Initial kernel scaffold; baseline (speedup 1.0000x reference)
import jax
import jax.numpy as jnp
from jax.experimental import pallas as pl


def kernel(x, W_enc, b_enc, W_dec, b_dec):
    raise NotImplementedError("write your pallas kernel here")


# speedup vs baseline: 12.2823x; 12.2823x over previous
import functools

import jax
import jax.numpy as jnp
from jax.experimental import pallas as pl

K = 64
_POS_INF_BITS = 0x7F800000


def _enc_body(x_ref, we_ref, be_ref, bd_ref, acts_ref):
    xb = x_ref[...] - bd_ref[...]
    prod = jax.lax.dot_general(
        xb, we_ref[...], (((1,), (1,)), ((), ())),
        preferred_element_type=jnp.float32)
    acts_ref[...] = jnp.maximum(prod + be_ref[...], 0.0)


def _encode(x, W_enc, b_enc, b_dec):
    Bm, D = x.shape
    F = W_enc.shape[0]
    BM = min(512, Bm)
    BF = min(2048, F)
    grid = (F // BF, Bm // BM)
    return pl.pallas_call(
        _enc_body,
        grid=grid,
        in_specs=[
            pl.BlockSpec((BM, D), lambda j, i: (i, 0)),
            pl.BlockSpec((BF, D), lambda j, i: (j, 0)),
            pl.BlockSpec((1, BF), lambda j, i: (0, j)),
            pl.BlockSpec((1, D), lambda j, i: (0, 0)),
        ],
        out_specs=pl.BlockSpec((BM, BF), lambda j, i: (i, j)),
        out_shape=jax.ShapeDtypeStruct((Bm, F), jnp.float32),
    )(x, W_enc, b_enc.reshape(1, F), b_dec.reshape(1, D))


def _thresh_body(acts_ref, th_ref):
    bits = jax.lax.bitcast_convert_type(acts_ref[...], jnp.int32)
    BM = bits.shape[0]

    def body(_, carry):
        lo, hi = carry
        mid = lo + ((hi - lo) >> 1)
        cnt = jnp.sum((bits >= mid).astype(jnp.int32), axis=1, keepdims=True)
        ge = cnt >= K
        return jnp.where(ge, mid, lo), jnp.where(ge, hi, mid)

    lo0 = jnp.zeros((BM, 1), jnp.int32)
    hi0 = jnp.full((BM, 1), _POS_INF_BITS, jnp.int32)
    lo, _ = jax.lax.fori_loop(0, 31, body, (lo0, hi0))
    th_ref[...] = jax.lax.bitcast_convert_type(lo, jnp.float32)


def _thresholds(acts):
    Bm, F = acts.shape
    BM = min(256, Bm)
    return pl.pallas_call(
        _thresh_body,
        grid=(Bm // BM,),
        in_specs=[pl.BlockSpec((BM, F), lambda i: (i, 0))],
        out_specs=pl.BlockSpec((BM, 1), lambda i: (i, 0)),
        out_shape=jax.ShapeDtypeStruct((Bm, 1), jnp.float32),
    )(acts)


def _dec_body(acts_ref, th_ref, wd_ref, bd_ref, out_ref):
    j = pl.program_id(1)
    a = acts_ref[...]
    feats = jnp.where(a >= th_ref[...], a, 0.0).astype(jnp.bfloat16)
    part = jax.lax.dot_general(
        feats, wd_ref[...], (((1,), (1,)), ((), ())),
        preferred_element_type=jnp.float32)

    @pl.when(j == 0)
    def _():
        out_ref[...] = bd_ref[...] + part

    @pl.when(j > 0)
    def _():
        out_ref[...] += part


def _decode(acts, thresh, W_dec, b_dec):
    Bm, F = acts.shape
    D = W_dec.shape[0]
    BM = min(512, Bm)
    BF = min(2048, F)
    wd16 = W_dec.astype(jnp.bfloat16)
    return pl.pallas_call(
        _dec_body,
        grid=(Bm // BM, F // BF),
        in_specs=[
            pl.BlockSpec((BM, BF), lambda i, j: (i, j)),
            pl.BlockSpec((BM, 1), lambda i, j: (i, 0)),
            pl.BlockSpec((D, BF), lambda i, j: (0, j)),
            pl.BlockSpec((1, D), lambda i, j: (0, 0)),
        ],
        out_specs=pl.BlockSpec((BM, D), lambda i, j: (i, 0)),
        out_shape=jax.ShapeDtypeStruct((Bm, D), jnp.float32),
    )(acts, thresh, wd16, b_dec.reshape(1, D))


def kernel(x, W_enc, b_enc, W_dec, b_dec):
    acts = _encode(x, W_enc, b_enc, b_dec)
    thresh = _thresholds(acts)
    return _decode(acts, thresh, W_dec, b_dec)
